# baseline (device time: 18114 ns/iter reference)
import jax
import jax.numpy as jnp
from jax import lax
from jax.experimental import pallas as pl
from jax.experimental.pallas import tpu as pltpu

K = 4


def kernel(x):
    m_per, n = x.shape
    mc = m_per // K

    def body(x_hbm, out_hbm, xv, send_buf, recv_buf,
             in_sems, my_out_sems, peer_out_sems, send_sems, recv_sems):
        my_x = lax.axis_index("x")
        my_y = lax.axis_index("y")
        my_z = lax.axis_index("z")
        partner = (1 - my_x, my_y, my_z)

        dmas_in = []
        for c in range(K):
            rows = pl.ds(c * mc, mc)
            dma = pltpu.make_async_copy(x_hbm.at[rows], xv.at[rows], in_sems.at[c])
            dma.start()
            dmas_in.append(dma)

        barrier_sem = pltpu.get_barrier_semaphore()
        pl.semaphore_signal(
            barrier_sem, inc=1,
            device_id=partner, device_id_type=pl.DeviceIdType.MESH,
        )
        pl.semaphore_wait(barrier_sem, 1)

        rdmas = []
        dmas_out = []
        for c in range(K):
            rows = pl.ds(c * mc, mc)
            dmas_in[c].wait()
            send_buf[rows, :] = xv[rows, :].astype(jnp.bfloat16)
            rdma = pltpu.make_async_remote_copy(
                src_ref=send_buf.at[rows],
                dst_ref=recv_buf.at[rows],
                send_sem=send_sems.at[c],
                recv_sem=recv_sems.at[c],
                device_id=partner,
                device_id_type=pl.DeviceIdType.MESH,
            )
            rdma.start()
            rdmas.append(rdma)
            out_rows = pl.ds(my_x * m_per + c * mc, mc)
            dma = pltpu.make_async_copy(
                send_buf.at[rows], out_hbm.at[out_rows], my_out_sems.at[c]
            )
            dma.start()
            dmas_out.append(dma)

        peer_dmas = []
        for c in range(K):
            rows = pl.ds(c * mc, mc)
            rdmas[c].wait_recv()
            out_rows = pl.ds((1 - my_x) * m_per + c * mc, mc)
            dma = pltpu.make_async_copy(
                recv_buf.at[rows], out_hbm.at[out_rows], peer_out_sems.at[c]
            )
            dma.start()
            peer_dmas.append(dma)

        for c in range(K):
            rdmas[c].wait_send()
            dmas_out[c].wait()
            peer_dmas[c].wait()

    return pl.pallas_call(
        body,
        out_shape=jax.ShapeDtypeStruct((2 * m_per, n), jnp.bfloat16),
        in_specs=[pl.BlockSpec(memory_space=pl.ANY)],
        out_specs=pl.BlockSpec(memory_space=pl.ANY),
        scratch_shapes=[
            pltpu.VMEM((m_per, n), x.dtype),
            pltpu.VMEM((m_per, n), jnp.bfloat16),
            pltpu.VMEM((m_per, n), jnp.bfloat16),
            pltpu.SemaphoreType.DMA((K,)),
            pltpu.SemaphoreType.DMA((K,)),
            pltpu.SemaphoreType.DMA((K,)),
            pltpu.SemaphoreType.DMA((K,)),
            pltpu.SemaphoreType.DMA((K,)),
        ],
        compiler_params=pltpu.CompilerParams(collective_id=0),
    )(x)


# device time: 17421 ns/iter; 1.0398x vs baseline; 1.0398x over previous
import jax
import jax.numpy as jnp
from jax import lax
from jax.experimental import pallas as pl
from jax.experimental.pallas import tpu as pltpu

K = 4


def kernel(x):
    m_per, n = x.shape
    mc = m_per // K

    def body(x_hbm, out_ref, xv, in_sems, send_sems, recv_sems):
        my_x = lax.axis_index("x")
        my_y = lax.axis_index("y")
        my_z = lax.axis_index("z")
        partner = (1 - my_x, my_y, my_z)

        dmas_in = []
        for c in range(K):
            rows = pl.ds(c * mc, mc)
            dma = pltpu.make_async_copy(x_hbm.at[rows], xv.at[rows], in_sems.at[c])
            dma.start()
            dmas_in.append(dma)

        barrier_sem = pltpu.get_barrier_semaphore()
        pl.semaphore_signal(
            barrier_sem, inc=1,
            device_id=partner, device_id_type=pl.DeviceIdType.MESH,
        )
        pl.semaphore_wait(barrier_sem, 1)

        rdmas = []
        for c in range(K):
            rows = pl.ds(c * mc, mc)
            out_rows = pl.ds(my_x * m_per + c * mc, mc)
            dmas_in[c].wait()
            out_ref[out_rows, :] = xv[rows, :].astype(jnp.bfloat16)
            rdma = pltpu.make_async_remote_copy(
                src_ref=out_ref.at[out_rows],
                dst_ref=out_ref.at[out_rows],
                send_sem=send_sems.at[c],
                recv_sem=recv_sems.at[c],
                device_id=partner,
                device_id_type=pl.DeviceIdType.MESH,
            )
            rdma.start()
            rdmas.append(rdma)

        for rdma in rdmas:
            rdma.wait()

    x = pltpu.with_memory_space_constraint(x, pltpu.MemorySpace.HBM)
    return pl.pallas_call(
        body,
        out_shape=jax.ShapeDtypeStruct((2 * m_per, n), jnp.bfloat16),
        in_specs=[pl.BlockSpec(memory_space=pltpu.MemorySpace.HBM)],
        out_specs=pl.BlockSpec(memory_space=pltpu.VMEM),
        scratch_shapes=[
            pltpu.VMEM((m_per, n), x.dtype),
            pltpu.SemaphoreType.DMA((K,)),
            pltpu.SemaphoreType.DMA((K,)),
            pltpu.SemaphoreType.DMA((K,)),
        ],
        compiler_params=pltpu.CompilerParams(collective_id=0),
    )(x)
